# baseline (speedup 1.0000x reference)
import functools

import jax
import jax.numpy as jnp
from jax import lax
from jax.experimental import pallas as pl
from jax.experimental.pallas import tpu as pltpu
from jax.experimental.pallas import tpu_sc as plsc


def _add_kernel(x_ref, pos_ref, out_ref):
    out_ref[...] = x_ref[...] + pos_ref[...][None, :, :]


def _kernel_tc(x, pos_table):
    B, L, D = x.shape
    BL = 512
    grid = (L // BL,)
    return pl.pallas_call(
        _add_kernel,
        grid=grid,
        in_specs=[
            pl.BlockSpec((B, BL, D), lambda l: (0, l, 0)),
            pl.BlockSpec((BL, D), lambda l: (l, 0)),
        ],
        out_specs=pl.BlockSpec((B, BL, D), lambda l: (0, l, 0)),
        out_shape=jax.ShapeDtypeStruct((B, L, D), x.dtype),
    )(x, pos_table)


def _kernel_tc_manual(x, pos_table):
    B, L, D = x.shape
    R = B * L
    CR = 256
    NB = 8
    NCH = R // CR
    n_rounds = NCH // NB

    xf = x.reshape(R, D)

    def body(x_hbm, pos_hbm, out_hbm, posv, xb, ob, insem, outsem, possem):
        pltpu.make_async_copy(
            pos_hbm.at[pl.ds(0, L)], posv, possem).start()

        def start_in(c, b):
            pltpu.make_async_copy(
                x_hbm.at[pl.ds(c * CR, CR)], xb.at[b], insem.at[b]).start()

        def wait_in(b):
            pltpu.make_async_copy(
                x_hbm.at[pl.ds(0, CR)], xb.at[b], insem.at[b]).wait()

        def start_out(c, b):
            pltpu.make_async_copy(
                ob.at[b], out_hbm.at[pl.ds(c * CR, CR)], outsem.at[b]).start()

        def wait_out(b):
            pltpu.make_async_copy(
                ob.at[b], out_hbm.at[pl.ds(0, CR)], outsem.at[b]).wait()

        for b in range(NB):
            start_in(b, b)
        pltpu.make_async_copy(pos_hbm.at[pl.ds(0, L)], posv, possem).wait()

        def round_body(r, _):
            for b in range(NB):
                c = r * NB + b
                wait_in(b)

                @pl.when(r >= 1)
                def _():
                    wait_out(b)

                poff = lax.rem(c * CR, L)
                ob[b, :, :] = xb[b, :, :] + posv[pl.ds(poff, CR), :]
                start_out(c, b)

                @pl.when(c + NB < NCH)
                def _():
                    start_in(c + NB, b)
            return 0

        lax.fori_loop(0, n_rounds, round_body, 0)
        for b in range(NB):
            wait_out(b)

    out = pl.pallas_call(
        body,
        in_specs=[
            pl.BlockSpec(memory_space=pl.ANY),
            pl.BlockSpec(memory_space=pl.ANY),
        ],
        out_specs=pl.BlockSpec(memory_space=pl.ANY),
        out_shape=jax.ShapeDtypeStruct((R, D), x.dtype),
        scratch_shapes=[
            pltpu.VMEM((L, D), jnp.float32),
            pltpu.VMEM((NB, CR, D), jnp.float32),
            pltpu.VMEM((NB, CR, D), jnp.float32),
            pltpu.SemaphoreType.DMA((NB,)),
            pltpu.SemaphoreType.DMA((NB,)),
            pltpu.SemaphoreType.DMA,
        ],
    )(xf, pos_table)
    return out.reshape(B, L, D)


def _kernel_sc(x, pos_table):
    B, L, D = x.shape
    R = B * L
    NC, NS = 2, 16
    NW = NC * NS
    rows_w = R // NW
    C = 16
    CD = C * D
    n_chunks = rows_w // C
    n_pairs = n_chunks // 2
    U = 8

    xf = x.reshape(R * D)
    posf = pos_table.reshape(-1)
    mesh = plsc.VectorSubcoreMesh(core_axis_name="c", subcore_axis_name="s")

    @functools.partial(
        pl.kernel,
        mesh=mesh,
        out_type=jax.ShapeDtypeStruct((R * D,), jnp.float32),
        scratch_types=[
            pltpu.VMEM((CD,), jnp.float32), pltpu.VMEM((CD,), jnp.float32),
            pltpu.VMEM((CD,), jnp.float32), pltpu.VMEM((CD,), jnp.float32),
            pltpu.VMEM((CD,), jnp.float32), pltpu.VMEM((CD,), jnp.float32),
            pltpu.SemaphoreType.DMA, pltpu.SemaphoreType.DMA,
            pltpu.SemaphoreType.DMA, pltpu.SemaphoreType.DMA,
            pltpu.SemaphoreType.DMA, pltpu.SemaphoreType.DMA,
        ],
    )
    def sc_add(x_hbm, pos_hbm, out_hbm,
               xb0, xb1, pb0, pb1, ob0, ob1,
               sx0, sx1, sp0, sp1, so0, so1):
        xb = (xb0, xb1)
        pb = (pb0, pb1)
        ob = (ob0, ob1)
        sx = (sx0, sx1)
        sp = (sp0, sp1)
        so = (so0, so1)
        wid = lax.axis_index("s") * NC + lax.axis_index("c")
        row0 = wid * rows_w
        prow0 = lax.rem(row0, L)

        def start_in(ci, b):
            base = (row0 + ci * C) * D
            pbase = (prow0 + ci * C) * D
            pltpu.async_copy(x_hbm.at[pl.ds(base, CD)], xb[b], sx[b])
            pltpu.async_copy(pos_hbm.at[pl.ds(pbase, CD)], pb[b], sp[b])

        def wait_in(b):
            pltpu.make_async_copy(x_hbm.at[pl.ds(0, CD)], xb[b], sx[b]).wait()
            pltpu.make_async_copy(pos_hbm.at[pl.ds(0, CD)], pb[b], sp[b]).wait()

        def start_out(ci, b):
            base = (row0 + ci * C) * D
            pltpu.async_copy(ob[b], out_hbm.at[pl.ds(base, CD)], so[b])

        def wait_out(b):
            pltpu.make_async_copy(ob[b], out_hbm.at[pl.ds(0, CD)], so[b]).wait()

        def compute(b):
            def vec(j, _):
                o = j * (16 * U)
                for u in range(U):
                    s = pl.ds(o + u * 16, 16)
                    ob[b][s] = xb[b][s] + pb[b][s]
                return 0
            lax.fori_loop(0, CD // (16 * U), vec, 0)

        start_in(0, 0)
        start_in(1, 1)

        def pair(pi, _):
            for b in range(2):
                ci = pi * 2 + b
                wait_in(b)

                @pl.when(pi >= 1)
                def _():
                    wait_out(b)

                compute(b)
                start_out(ci, b)

                @pl.when(ci + 2 < n_chunks)
                def _():
                    start_in(ci + 2, b)
            return 0

        lax.fori_loop(0, n_pairs, pair, 0)
        wait_out(0)
        wait_out(1)

    out = sc_add(xf, posf)
    return out.reshape(B, L, D)


kernel = _kernel_tc_manual


# speedup vs baseline: 1.0054x; 1.0054x over previous
import functools

import jax
import jax.numpy as jnp
from jax import lax
from jax.experimental import pallas as pl
from jax.experimental.pallas import tpu as pltpu
from jax.experimental.pallas import tpu_sc as plsc


def _add_kernel(x_ref, pos_ref, out_ref):
    out_ref[...] = x_ref[...] + pos_ref[...][None, :, :]


def _kernel_tc(x, pos_table):
    B, L, D = x.shape
    BL = 512
    grid = (L // BL,)
    return pl.pallas_call(
        _add_kernel,
        grid=grid,
        in_specs=[
            pl.BlockSpec((B, BL, D), lambda l: (0, l, 0)),
            pl.BlockSpec((BL, D), lambda l: (l, 0)),
        ],
        out_specs=pl.BlockSpec((B, BL, D), lambda l: (0, l, 0)),
        out_shape=jax.ShapeDtypeStruct((B, L, D), x.dtype),
    )(x, pos_table)


def _kernel_tc_manual(x, pos_table):
    B, L, D = x.shape
    R = B * L
    CR = 256
    NB = 8
    NCH = R // CR
    n_rounds = NCH // NB

    xf = x.reshape(R, D)

    NP = L // CR

    def body(x_hbm, pos_hbm, out_hbm, posv, xb, ob, insem, outsem, possem):
        def start_in(c, b):
            pltpu.make_async_copy(
                x_hbm.at[pl.ds(c * CR, CR)], xb.at[b], insem.at[b]).start()

        def wait_in(b):
            pltpu.make_async_copy(
                x_hbm.at[pl.ds(0, CR)], xb.at[b], insem.at[b]).wait()

        def start_out(c, b):
            pltpu.make_async_copy(
                ob.at[b], out_hbm.at[pl.ds(c * CR, CR)], outsem.at[b]).start()

        def wait_out(b):
            pltpu.make_async_copy(
                ob.at[b], out_hbm.at[pl.ds(0, CR)], outsem.at[b]).wait()

        for p in range(NP):
            pltpu.make_async_copy(
                pos_hbm.at[pl.ds(p * CR, CR)],
                posv.at[pl.ds(p * CR, CR)], possem.at[p]).start()
            if p < NB:
                start_in(p, p)

        def round_body(r, _):
            for b in range(NB):
                c = r * NB + b

                @pl.when(c < NP)
                def _():
                    poff = c * CR
                    pltpu.make_async_copy(
                        pos_hbm.at[pl.ds(poff, CR)],
                        posv.at[pl.ds(poff, CR)], possem.at[c]).wait()

                wait_in(b)

                @pl.when(r >= 1)
                def _():
                    wait_out(b)

                poff = lax.rem(c * CR, L)
                ob[b, :, :] = xb[b, :, :] + posv[pl.ds(poff, CR), :]
                start_out(c, b)

                @pl.when(c + NB < NCH)
                def _():
                    start_in(c + NB, b)
            return 0

        lax.fori_loop(0, n_rounds, round_body, 0)
        for b in range(NB):
            wait_out(b)

    out = pl.pallas_call(
        body,
        in_specs=[
            pl.BlockSpec(memory_space=pl.ANY),
            pl.BlockSpec(memory_space=pl.ANY),
        ],
        out_specs=pl.BlockSpec(memory_space=pl.ANY),
        out_shape=jax.ShapeDtypeStruct((R, D), x.dtype),
        scratch_shapes=[
            pltpu.VMEM((L, D), jnp.float32),
            pltpu.VMEM((NB, CR, D), jnp.float32),
            pltpu.VMEM((NB, CR, D), jnp.float32),
            pltpu.SemaphoreType.DMA((NB,)),
            pltpu.SemaphoreType.DMA((NB,)),
            pltpu.SemaphoreType.DMA((L // CR,)),
        ],
    )(xf, pos_table)
    return out.reshape(B, L, D)


def _kernel_sc(x, pos_table):
    B, L, D = x.shape
    R = B * L
    NC, NS = 2, 16
    NW = NC * NS
    rows_w = R // NW
    C = 16
    CD = C * D
    n_chunks = rows_w // C
    n_pairs = n_chunks // 2
    U = 8

    xf = x.reshape(R * D)
    posf = pos_table.reshape(-1)
    mesh = plsc.VectorSubcoreMesh(core_axis_name="c", subcore_axis_name="s")

    @functools.partial(
        pl.kernel,
        mesh=mesh,
        out_type=jax.ShapeDtypeStruct((R * D,), jnp.float32),
        scratch_types=[
            pltpu.VMEM((CD,), jnp.float32), pltpu.VMEM((CD,), jnp.float32),
            pltpu.VMEM((CD,), jnp.float32), pltpu.VMEM((CD,), jnp.float32),
            pltpu.VMEM((CD,), jnp.float32), pltpu.VMEM((CD,), jnp.float32),
            pltpu.SemaphoreType.DMA, pltpu.SemaphoreType.DMA,
            pltpu.SemaphoreType.DMA, pltpu.SemaphoreType.DMA,
            pltpu.SemaphoreType.DMA, pltpu.SemaphoreType.DMA,
        ],
    )
    def sc_add(x_hbm, pos_hbm, out_hbm,
               xb0, xb1, pb0, pb1, ob0, ob1,
               sx0, sx1, sp0, sp1, so0, so1):
        xb = (xb0, xb1)
        pb = (pb0, pb1)
        ob = (ob0, ob1)
        sx = (sx0, sx1)
        sp = (sp0, sp1)
        so = (so0, so1)
        wid = lax.axis_index("s") * NC + lax.axis_index("c")
        row0 = wid * rows_w
        prow0 = lax.rem(row0, L)

        def start_in(ci, b):
            base = (row0 + ci * C) * D
            pbase = (prow0 + ci * C) * D
            pltpu.async_copy(x_hbm.at[pl.ds(base, CD)], xb[b], sx[b])
            pltpu.async_copy(pos_hbm.at[pl.ds(pbase, CD)], pb[b], sp[b])

        def wait_in(b):
            pltpu.make_async_copy(x_hbm.at[pl.ds(0, CD)], xb[b], sx[b]).wait()
            pltpu.make_async_copy(pos_hbm.at[pl.ds(0, CD)], pb[b], sp[b]).wait()

        def start_out(ci, b):
            base = (row0 + ci * C) * D
            pltpu.async_copy(ob[b], out_hbm.at[pl.ds(base, CD)], so[b])

        def wait_out(b):
            pltpu.make_async_copy(ob[b], out_hbm.at[pl.ds(0, CD)], so[b]).wait()

        def compute(b):
            def vec(j, _):
                o = j * (16 * U)
                for u in range(U):
                    s = pl.ds(o + u * 16, 16)
                    ob[b][s] = xb[b][s] + pb[b][s]
                return 0
            lax.fori_loop(0, CD // (16 * U), vec, 0)

        start_in(0, 0)
        start_in(1, 1)

        def pair(pi, _):
            for b in range(2):
                ci = pi * 2 + b
                wait_in(b)

                @pl.when(pi >= 1)
                def _():
                    wait_out(b)

                compute(b)
                start_out(ci, b)

                @pl.when(ci + 2 < n_chunks)
                def _():
                    start_in(ci + 2, b)
            return 0

        lax.fori_loop(0, n_pairs, pair, 0)
        wait_out(0)
        wait_out(1)

    out = sc_add(xf, posf)
    return out.reshape(B, L, D)


kernel = _kernel_tc_manual


# speedup vs baseline: 1.0095x; 1.0041x over previous
import functools

import jax
import jax.numpy as jnp
from jax import lax
from jax.experimental import pallas as pl
from jax.experimental.pallas import tpu as pltpu
from jax.experimental.pallas import tpu_sc as plsc


def _add_kernel(x_ref, pos_ref, out_ref):
    out_ref[...] = x_ref[...] + pos_ref[...][None, :, :]


def _kernel_tc(x, pos_table):
    B, L, D = x.shape
    BL = 512
    grid = (L // BL,)
    return pl.pallas_call(
        _add_kernel,
        grid=grid,
        in_specs=[
            pl.BlockSpec((B, BL, D), lambda l: (0, l, 0)),
            pl.BlockSpec((BL, D), lambda l: (l, 0)),
        ],
        out_specs=pl.BlockSpec((B, BL, D), lambda l: (0, l, 0)),
        out_shape=jax.ShapeDtypeStruct((B, L, D), x.dtype),
    )(x, pos_table)


def _kernel_tc_manual(x, pos_table):
    B, L, D = x.shape
    R = B * L
    CR = 1024
    NB = 4
    NCH = R // CR
    n_rounds = NCH // NB

    xf = x.reshape(R, D)

    NP = L // CR

    def body(x_hbm, pos_hbm, out_hbm, posv, xb, ob, insem, outsem, possem):
        def start_in(c, b):
            pltpu.make_async_copy(
                x_hbm.at[pl.ds(c * CR, CR)], xb.at[b], insem.at[b]).start()

        def wait_in(b):
            pltpu.make_async_copy(
                x_hbm.at[pl.ds(0, CR)], xb.at[b], insem.at[b]).wait()

        def start_out(c, b):
            pltpu.make_async_copy(
                ob.at[b], out_hbm.at[pl.ds(c * CR, CR)], outsem.at[b]).start()

        def wait_out(b):
            pltpu.make_async_copy(
                ob.at[b], out_hbm.at[pl.ds(0, CR)], outsem.at[b]).wait()

        for p in range(NP):
            pltpu.make_async_copy(
                pos_hbm.at[pl.ds(p * CR, CR)],
                posv.at[pl.ds(p * CR, CR)], possem.at[p]).start()
            if p < NB:
                start_in(p, p)

        def round_body(r, _):
            for b in range(NB):
                c = r * NB + b

                @pl.when(c < NP)
                def _():
                    poff = c * CR
                    pltpu.make_async_copy(
                        pos_hbm.at[pl.ds(poff, CR)],
                        posv.at[pl.ds(poff, CR)], possem.at[c]).wait()

                wait_in(b)

                @pl.when(r >= 1)
                def _():
                    wait_out(b)

                poff = lax.rem(c * CR, L)
                ob[b, :, :] = xb[b, :, :] + posv[pl.ds(poff, CR), :]
                start_out(c, b)

                @pl.when(c + NB < NCH)
                def _():
                    start_in(c + NB, b)
            return 0

        lax.fori_loop(0, n_rounds, round_body, 0)
        for b in range(NB):
            wait_out(b)

    out = pl.pallas_call(
        body,
        in_specs=[
            pl.BlockSpec(memory_space=pl.ANY),
            pl.BlockSpec(memory_space=pl.ANY),
        ],
        out_specs=pl.BlockSpec(memory_space=pl.ANY),
        out_shape=jax.ShapeDtypeStruct((R, D), x.dtype),
        scratch_shapes=[
            pltpu.VMEM((L, D), jnp.float32),
            pltpu.VMEM((NB, CR, D), jnp.float32),
            pltpu.VMEM((NB, CR, D), jnp.float32),
            pltpu.SemaphoreType.DMA((NB,)),
            pltpu.SemaphoreType.DMA((NB,)),
            pltpu.SemaphoreType.DMA((L // CR,)),
        ],
    )(xf, pos_table)
    return out.reshape(B, L, D)


def _kernel_sc(x, pos_table):
    B, L, D = x.shape
    R = B * L
    NC, NS = 2, 16
    NW = NC * NS
    rows_w = R // NW
    C = 16
    CD = C * D
    n_chunks = rows_w // C
    n_pairs = n_chunks // 2
    U = 8

    xf = x.reshape(R * D)
    posf = pos_table.reshape(-1)
    mesh = plsc.VectorSubcoreMesh(core_axis_name="c", subcore_axis_name="s")

    @functools.partial(
        pl.kernel,
        mesh=mesh,
        out_type=jax.ShapeDtypeStruct((R * D,), jnp.float32),
        scratch_types=[
            pltpu.VMEM((CD,), jnp.float32), pltpu.VMEM((CD,), jnp.float32),
            pltpu.VMEM((CD,), jnp.float32), pltpu.VMEM((CD,), jnp.float32),
            pltpu.VMEM((CD,), jnp.float32), pltpu.VMEM((CD,), jnp.float32),
            pltpu.SemaphoreType.DMA, pltpu.SemaphoreType.DMA,
            pltpu.SemaphoreType.DMA, pltpu.SemaphoreType.DMA,
            pltpu.SemaphoreType.DMA, pltpu.SemaphoreType.DMA,
        ],
    )
    def sc_add(x_hbm, pos_hbm, out_hbm,
               xb0, xb1, pb0, pb1, ob0, ob1,
               sx0, sx1, sp0, sp1, so0, so1):
        xb = (xb0, xb1)
        pb = (pb0, pb1)
        ob = (ob0, ob1)
        sx = (sx0, sx1)
        sp = (sp0, sp1)
        so = (so0, so1)
        wid = lax.axis_index("s") * NC + lax.axis_index("c")
        row0 = wid * rows_w
        prow0 = lax.rem(row0, L)

        def start_in(ci, b):
            base = (row0 + ci * C) * D
            pbase = (prow0 + ci * C) * D
            pltpu.async_copy(x_hbm.at[pl.ds(base, CD)], xb[b], sx[b])
            pltpu.async_copy(pos_hbm.at[pl.ds(pbase, CD)], pb[b], sp[b])

        def wait_in(b):
            pltpu.make_async_copy(x_hbm.at[pl.ds(0, CD)], xb[b], sx[b]).wait()
            pltpu.make_async_copy(pos_hbm.at[pl.ds(0, CD)], pb[b], sp[b]).wait()

        def start_out(ci, b):
            base = (row0 + ci * C) * D
            pltpu.async_copy(ob[b], out_hbm.at[pl.ds(base, CD)], so[b])

        def wait_out(b):
            pltpu.make_async_copy(ob[b], out_hbm.at[pl.ds(0, CD)], so[b]).wait()

        def compute(b):
            def vec(j, _):
                o = j * (16 * U)
                for u in range(U):
                    s = pl.ds(o + u * 16, 16)
                    ob[b][s] = xb[b][s] + pb[b][s]
                return 0
            lax.fori_loop(0, CD // (16 * U), vec, 0)

        start_in(0, 0)
        start_in(1, 1)

        def pair(pi, _):
            for b in range(2):
                ci = pi * 2 + b
                wait_in(b)

                @pl.when(pi >= 1)
                def _():
                    wait_out(b)

                compute(b)
                start_out(ci, b)

                @pl.when(ci + 2 < n_chunks)
                def _():
                    start_in(ci + 2, b)
            return 0

        lax.fori_loop(0, n_pairs, pair, 0)
        wait_out(0)
        wait_out(1)

    out = sc_add(xf, posf)
    return out.reshape(B, L, D)


kernel = _kernel_tc_manual


# speedup vs baseline: 1.0138x; 1.0043x over previous
import jax
import jax.numpy as jnp
from jax import lax
from jax.experimental import pallas as pl
from jax.experimental.pallas import tpu as pltpu


def kernel(x, pos_table):
    B, L, D = x.shape
    R = B * L
    CR = 1024
    NB = 4
    NCH = R // CR
    n_rounds = NCH // NB
    NP = L // CR

    xf = x.reshape(R, D)

    def body(x_hbm, pos_hbm, out_hbm, posv, xb, ob, insem, outsem, possem):
        def start_in(c, b):
            pltpu.make_async_copy(
                x_hbm.at[pl.ds(c * CR, CR)], xb.at[b], insem.at[b]).start()

        def wait_in(b):
            pltpu.make_async_copy(
                x_hbm.at[pl.ds(0, CR)], xb.at[b], insem.at[b]).wait()

        def start_out(c, b):
            pltpu.make_async_copy(
                ob.at[b], out_hbm.at[pl.ds(c * CR, CR)], outsem.at[b]).start()

        def wait_out(b):
            pltpu.make_async_copy(
                ob.at[b], out_hbm.at[pl.ds(0, CR)], outsem.at[b]).wait()

        for p in range(NP):
            pltpu.make_async_copy(
                pos_hbm.at[pl.ds(p * CR, CR)],
                posv.at[pl.ds(p * CR, CR)], possem.at[p]).start()
            if p < NB:
                start_in(p, p)

        def round_body(r, _):
            for b in range(NB):
                c = r * NB + b

                @pl.when(c < NP)
                def _():
                    poff = c * CR
                    pltpu.make_async_copy(
                        pos_hbm.at[pl.ds(poff, CR)],
                        posv.at[pl.ds(poff, CR)], possem.at[c]).wait()

                wait_in(b)

                @pl.when(r >= 1)
                def _():
                    wait_out(b)

                poff = lax.rem(c * CR, L)
                ob[b, :, :] = xb[b, :, :] + posv[pl.ds(poff, CR), :]
                start_out(c, b)

                @pl.when(c + NB < NCH)
                def _():
                    start_in(c + NB, b)
            return 0

        lax.fori_loop(0, n_rounds, round_body, 0)
        for b in range(NB):
            wait_out(b)

    out = pl.pallas_call(
        body,
        in_specs=[
            pl.BlockSpec(memory_space=pl.ANY),
            pl.BlockSpec(memory_space=pl.ANY),
        ],
        out_specs=pl.BlockSpec(memory_space=pl.ANY),
        out_shape=jax.ShapeDtypeStruct((R, D), x.dtype),
        scratch_shapes=[
            pltpu.VMEM((L, D), jnp.float32),
            pltpu.VMEM((NB, CR, D), jnp.float32),
            pltpu.VMEM((NB, CR, D), jnp.float32),
            pltpu.SemaphoreType.DMA((NB,)),
            pltpu.SemaphoreType.DMA((NB,)),
            pltpu.SemaphoreType.DMA((NP,)),
        ],
    )(xf, pos_table)
    return out.reshape(B, L, D)
